# Initial kernel scaffold; baseline (speedup 1.0000x reference)
import jax
import jax.numpy as jnp
from jax.experimental import pallas as pl


def kernel(z, W1, b1, Wg, bg, W2, b2, edge_index):
    raise NotImplementedError("write your pallas kernel here")


# speedup vs baseline: 28.7076x; 28.7076x over previous
import functools

import jax
import jax.numpy as jnp
from jax import lax
from jax.experimental import pallas as pl
from jax.experimental.pallas import tpu as pltpu
from jax.experimental.pallas import tpu_sc as plsc

N = 10000
D = 128
E = 320000

NC = 2
NS = 16
NW = NC * NS
CK = 128
CW = 80
EP = NW * CW * CK
NP = 10240
RPT = NP // NS
GRID = 8
RB = NP // GRID
HW = 64

_mesh = plsc.VectorSubcoreMesh(
    core_axis_name="c", subcore_axis_name="s", num_cores=NC, num_subcores=NS
)


def _k1_body(z_ref, w1_ref, b1_ref, wg_ref, o_ref):
    h = jnp.dot(z_ref[...], w1_ref[...], preferred_element_type=jnp.float32)
    h = jnp.maximum(h + b1_ref[...], 0.0)
    o_ref[...] = jnp.dot(h, wg_ref[...], preferred_element_type=jnp.float32)


def _k1(z_p, W1, b1r, Wg):
    return pl.pallas_call(
        _k1_body,
        grid=(GRID,),
        in_specs=[
            pl.BlockSpec((RB, D), lambda i: (i, 0)),
            pl.BlockSpec((D, D), lambda i: (0, 0)),
            pl.BlockSpec((1, D), lambda i: (0, 0)),
            pl.BlockSpec((D, D), lambda i: (0, 0)),
        ],
        out_specs=pl.BlockSpec((RB, D), lambda i: (i, 0)),
        out_shape=jax.ShapeDtypeStruct((NP, D), jnp.float32),
    )(z_p, W1, b1r, Wg)


@functools.partial(
    pl.kernel,
    out_type=jax.ShapeDtypeStruct((NC, NP, 16), jnp.float32),
    mesh=_mesh,
    compiler_params=pltpu.CompilerParams(use_tc_tiling_on_sc=False),
    scratch_types=[
        pltpu.VMEM((CW, CK), jnp.int32),
        pltpu.VMEM((CK, 16), jnp.float32),
        pltpu.VMEM((RPT, 16), jnp.float32),
        pltpu.VMEM_SHARED((NP, 16), jnp.float32),
        pltpu.SemaphoreType.DMA,
    ],
)
def _deg_kernel(d_hbm, oh_hbm, z16_hbm, out_hbm, dv, oh, zb, acc, sem):
    cid = lax.axis_index("c")
    sid = lax.axis_index("s")
    wid = sid * NC + cid
    base = sid * RPT
    pltpu.sync_copy(d_hbm.at[wid], dv)
    pltpu.sync_copy(oh_hbm, oh)
    pltpu.sync_copy(z16_hbm, zb)
    pltpu.sync_copy(zb, acc.at[pl.ds(base, RPT)])
    plsc.subcore_barrier()

    def _start(j, carry):
        pltpu.async_copy(oh, acc.at[dv.at[j]], sem, add=True)
        return carry

    lax.fori_loop(0, CW, _start, 0)

    def _drain(j, carry):
        pltpu.make_async_copy(oh, acc.at[dv.at[0]], sem).wait()
        return carry

    lax.fori_loop(0, CW, _drain, 0)
    plsc.subcore_barrier()
    pltpu.sync_copy(acc.at[pl.ds(base, RPT)], zb)
    pltpu.sync_copy(zb, out_hbm.at[cid, pl.ds(base, RPT)])


def _k3_body(xw_ref, d0_ref, d1_ref, ylo_ref, yhi_ref):
    deg = d0_ref[:, :1] + d1_ref[:, :1] + 1.0
    y = xw_ref[...] * lax.rsqrt(deg)
    ylo_ref[...] = y[:, :HW]
    yhi_ref[...] = y[:, HW:]


def _k3(xw, deg0, deg1):
    return pl.pallas_call(
        _k3_body,
        grid=(GRID,),
        in_specs=[
            pl.BlockSpec((RB, D), lambda i: (i, 0)),
            pl.BlockSpec((RB, 16), lambda i: (i, 0)),
            pl.BlockSpec((RB, 16), lambda i: (i, 0)),
        ],
        out_specs=[pl.BlockSpec((RB, HW), lambda i: (i, 0)),
                   pl.BlockSpec((RB, HW), lambda i: (i, 0))],
        out_shape=[jax.ShapeDtypeStruct((NP, HW), jnp.float32),
                   jax.ShapeDtypeStruct((NP, HW), jnp.float32)],
    )(xw, deg0, deg1)


@functools.partial(
    pl.kernel,
    out_type=jax.ShapeDtypeStruct((NC, NP, HW), jnp.float32),
    mesh=_mesh,
    compiler_params=pltpu.CompilerParams(use_tc_tiling_on_sc=False),
    scratch_types=[
        pltpu.VMEM((CW, CK), jnp.int32),
        pltpu.VMEM((CW, CK), jnp.int32),
        pltpu.VMEM((CK, HW), jnp.float32),
        pltpu.VMEM((CK, HW), jnp.float32),
        pltpu.VMEM_SHARED((NP, HW), jnp.float32),
        pltpu.SemaphoreType.DMA,
        pltpu.SemaphoreType.DMA,
        pltpu.SemaphoreType.DMA,
    ],
)
def _seg_kernel(y_hbm, s_hbm, d_hbm, z128_hbm, out_hbm,
                sv, dv, rb0, rb1, acc, g0, g1, ss):
    cid = lax.axis_index("c")
    sid = lax.axis_index("s")
    wid = sid * NC + cid
    base = sid * RPT
    pltpu.sync_copy(s_hbm.at[wid], sv)
    pltpu.sync_copy(d_hbm.at[wid], dv)

    for k in range(RPT // CK):
        @pl.when(cid == 0)
        def _():
            pltpu.sync_copy(y_hbm.at[pl.ds(base + k * CK, CK)], rb1)

        @pl.when(cid != 0)
        def _():
            pltpu.sync_copy(z128_hbm, rb1)

        pltpu.sync_copy(rb1, acc.at[pl.ds(base + k * CK, CK)])
    plsc.subcore_barrier()

    rbs = (rb0, rb1)
    gs = (g0, g1)
    pltpu.async_copy(y_hbm.at[sv.at[0]], rb0, g0)
    pltpu.async_copy(y_hbm.at[sv.at[1]], rb1, g1)

    def _body(t, carry):
        for b in range(2):
            j = 2 * t + b
            pltpu.make_async_copy(y_hbm.at[sv.at[j]], rbs[b], gs[b]).wait()
            pltpu.async_copy(rbs[b], acc.at[dv.at[j]], ss, add=True).wait()
            pltpu.async_copy(y_hbm.at[sv.at[j + 2]], rbs[b], gs[b])
        return carry

    lax.fori_loop(0, CW // 2 - 1, _body, 0)
    for b in range(2):
        j = CW - 2 + b
        pltpu.make_async_copy(y_hbm.at[sv.at[j]], rbs[b], gs[b]).wait()
        pltpu.async_copy(rbs[b], acc.at[dv.at[j]], ss, add=True).wait()
    plsc.subcore_barrier()

    for k in range(RPT // CK):
        pltpu.sync_copy(acc.at[pl.ds(base + k * CK, CK)], rb0)
        pltpu.sync_copy(rb0, out_hbm.at[cid, pl.ds(base + k * CK, CK)])


def _k5_body(plo0_ref, plo1_ref, phi0_ref, phi1_ref, d0_ref, d1_ref,
             bg_ref, w2_ref, b2_ref, o_ref):
    deg = d0_ref[:, :1] + d1_ref[:, :1] + 1.0
    dis = lax.rsqrt(deg)
    hlo = jnp.maximum((plo0_ref[...] + plo1_ref[...]) * dis + bg_ref[:, :HW], 0.0)
    hhi = jnp.maximum((phi0_ref[...] + phi1_ref[...]) * dis + bg_ref[:, HW:], 0.0)
    o = jnp.dot(hlo, w2_ref[:HW, :], preferred_element_type=jnp.float32)
    o = o + jnp.dot(hhi, w2_ref[HW:, :], preferred_element_type=jnp.float32)
    o = o + b2_ref[...]
    col = lax.broadcasted_iota(jnp.int32, (RB, D), 1)
    o_ref[...] = jnp.where(col == 0, jax.nn.sigmoid(o), o)


def _k5(plo0, plo1, phi0, phi1, deg0, deg1, bgr, W2, b2r):
    return pl.pallas_call(
        _k5_body,
        grid=(GRID,),
        in_specs=[
            pl.BlockSpec((RB, HW), lambda i: (i, 0)),
            pl.BlockSpec((RB, HW), lambda i: (i, 0)),
            pl.BlockSpec((RB, HW), lambda i: (i, 0)),
            pl.BlockSpec((RB, HW), lambda i: (i, 0)),
            pl.BlockSpec((RB, 16), lambda i: (i, 0)),
            pl.BlockSpec((RB, 16), lambda i: (i, 0)),
            pl.BlockSpec((1, D), lambda i: (0, 0)),
            pl.BlockSpec((D, D), lambda i: (0, 0)),
            pl.BlockSpec((1, D), lambda i: (0, 0)),
        ],
        out_specs=pl.BlockSpec((RB, D), lambda i: (i, 0)),
        out_shape=jax.ShapeDtypeStruct((NP, D), jnp.float32),
    )(plo0, plo1, phi0, phi1, deg0, deg1, bgr, W2, b2r)


@jax.jit
def kernel(z, W1, b1, Wg, bg, W2, b2, edge_index):
    z_p = jnp.pad(z, ((0, NP - N), (0, 0)))
    b1r = b1.reshape(1, D)
    bgr = bg.reshape(1, D)
    b2r = b2.reshape(1, D)

    npad = EP - E
    pad_idx = (N + (jnp.arange(npad, dtype=jnp.int32) % (NP - N))).astype(jnp.int32)
    s_r = jnp.concatenate([edge_index[0], pad_idx]).reshape(NW, CW, CK)
    d_r = jnp.concatenate([edge_index[1], pad_idx]).reshape(NW, CW, CK)

    onehot = jnp.zeros((CK, 16), jnp.float32).at[:, 0].set(1.0)
    zeros16 = jnp.zeros((RPT, 16), jnp.float32)
    zerosrow = jnp.zeros((CK, HW), jnp.float32)

    xw = _k1(z_p, W1, b1r, Wg)
    degp = _deg_kernel(d_r, onehot, zeros16)
    deg0, deg1 = degp[0], degp[1]
    ylo, yhi = _k3(xw, deg0, deg1)
    plo = _seg_kernel(ylo, s_r, d_r, zerosrow)
    phi = _seg_kernel(yhi, s_r, d_r, zerosrow)
    out = _k5(plo[0], plo[1], phi[0], phi[1], deg0, deg1, bgr, W2, b2r)
    return out[:N]


# speedup vs baseline: 31.3049x; 1.0905x over previous
import functools

import jax
import jax.numpy as jnp
from jax import lax
from jax.experimental import pallas as pl
from jax.experimental.pallas import tpu as pltpu
from jax.experimental.pallas import tpu_sc as plsc

N = 10000
D = 128
E = 320000

NC = 2
NS = 16
NW = NC * NS
CK = 128
CW = 80
EP = NW * CW * CK
NP = 10240
RPT = NP // NS
HW = 64
GRID = 8
RB = NP // GRID
GRID_O = 10
RBO = N // GRID_O

_mesh = plsc.VectorSubcoreMesh(
    core_axis_name="c", subcore_axis_name="s", num_cores=NC, num_subcores=NS
)
_sc_params = pltpu.CompilerParams(use_tc_tiling_on_sc=False)


@functools.partial(
    pl.kernel,
    out_type=[jax.ShapeDtypeStruct((NP, 16), jnp.float32),
              jax.ShapeDtypeStruct((NP, 16), jnp.float32)],
    mesh=_mesh,
    compiler_params=_sc_params,
    scratch_types=[
        pltpu.VMEM((CW, CK), jnp.int32),
        pltpu.VMEM((CK, 16), jnp.float32),
        pltpu.VMEM((RPT, 16), jnp.float32),
        pltpu.VMEM_SHARED((NP, 16), jnp.float32),
        pltpu.SemaphoreType.DMA,
    ],
)
def _deg_kernel(d_hbm, oh_hbm, z16_hbm, out0_hbm, out1_hbm, dv, oh, zb, acc, sem):
    cid = lax.axis_index("c")
    sid = lax.axis_index("s")
    wid = sid * NC + cid
    base = sid * RPT
    pltpu.sync_copy(d_hbm.at[wid], dv)
    pltpu.sync_copy(oh_hbm, oh)
    pltpu.sync_copy(z16_hbm, zb)
    pltpu.sync_copy(zb, acc.at[pl.ds(base, RPT)])
    plsc.subcore_barrier()

    def _start(j, carry):
        pltpu.async_copy(oh, acc.at[dv.at[j]], sem, add=True)
        return carry

    lax.fori_loop(0, CW, _start, 0)

    def _drain(j, carry):
        pltpu.make_async_copy(oh, acc.at[dv.at[0]], sem).wait()
        return carry

    lax.fori_loop(0, CW, _drain, 0)
    plsc.subcore_barrier()
    pltpu.sync_copy(acc.at[pl.ds(base, RPT)], zb)

    @pl.when(cid == 0)
    def _():
        pltpu.sync_copy(zb, out0_hbm.at[pl.ds(base, RPT)])

    @pl.when(cid != 0)
    def _():
        pltpu.sync_copy(zb, out1_hbm.at[pl.ds(base, RPT)])


def _ka_body(z_ref, w1_ref, b1_ref, wg_ref, d0_ref, d1_ref, ylo_ref, yhi_ref):
    h = jnp.dot(z_ref[...], w1_ref[...], preferred_element_type=jnp.float32)
    h = jnp.maximum(h + b1_ref[...], 0.0)
    xw = jnp.dot(h, wg_ref[...], preferred_element_type=jnp.float32)
    deg = d0_ref[:, :1] + d1_ref[:, :1] + 1.0
    y = xw * lax.rsqrt(deg)
    ylo_ref[...] = y[:, :HW]
    yhi_ref[...] = y[:, HW:]


def _ka(z_p, W1, b1r, Wg, deg0, deg1):
    return pl.pallas_call(
        _ka_body,
        grid=(GRID,),
        in_specs=[
            pl.BlockSpec((RB, D), lambda i: (i, 0)),
            pl.BlockSpec((D, D), lambda i: (0, 0)),
            pl.BlockSpec((1, D), lambda i: (0, 0)),
            pl.BlockSpec((D, D), lambda i: (0, 0)),
            pl.BlockSpec((RB, 16), lambda i: (i, 0)),
            pl.BlockSpec((RB, 16), lambda i: (i, 0)),
        ],
        out_specs=[pl.BlockSpec((RB, HW), lambda i: (i, 0)),
                   pl.BlockSpec((RB, HW), lambda i: (i, 0))],
        out_shape=[jax.ShapeDtypeStruct((NP, HW), jnp.float32),
                   jax.ShapeDtypeStruct((NP, HW), jnp.float32)],
    )(z_p, W1, b1r, Wg, deg0, deg1)


@functools.partial(
    pl.kernel,
    out_type=[jax.ShapeDtypeStruct((NP, HW), jnp.float32)] * 4,
    mesh=_mesh,
    compiler_params=_sc_params,
    scratch_types=[
        pltpu.VMEM((CW, CK), jnp.int32),
        pltpu.VMEM((CW, CK), jnp.int32),
        pltpu.VMEM((CK, HW), jnp.float32),
        pltpu.VMEM((CK, HW), jnp.float32),
        pltpu.VMEM_SHARED((NP, HW), jnp.float32),
        pltpu.SemaphoreType.DMA,
        pltpu.SemaphoreType.DMA,
        pltpu.SemaphoreType.DMA,
    ],
)
def _seg_kernel(ylo_hbm, yhi_hbm, s_hbm, d_hbm, zrow_hbm,
                p0lo_hbm, p1lo_hbm, p0hi_hbm, p1hi_hbm,
                sv, dv, rb0, rb1, acc, g0, g1, ss):
    cid = lax.axis_index("c")
    sid = lax.axis_index("s")
    wid = sid * NC + cid
    base = sid * RPT
    pltpu.sync_copy(s_hbm.at[wid], sv)
    pltpu.sync_copy(d_hbm.at[wid], dv)
    rbs = (rb0, rb1)
    gs = (g0, g1)

    for y_hbm, o0_hbm, o1_hbm in ((ylo_hbm, p0lo_hbm, p1lo_hbm),
                                  (yhi_hbm, p0hi_hbm, p1hi_hbm)):
        for k in range(RPT // CK):
            @pl.when(cid == 0)
            def _():
                pltpu.sync_copy(y_hbm.at[pl.ds(base + k * CK, CK)], rb1)

            @pl.when(cid != 0)
            def _():
                pltpu.sync_copy(zrow_hbm, rb1)

            pltpu.sync_copy(rb1, acc.at[pl.ds(base + k * CK, CK)])
        plsc.subcore_barrier()

        pltpu.async_copy(y_hbm.at[sv.at[0]], rb0, g0)
        pltpu.async_copy(y_hbm.at[sv.at[1]], rb1, g1)

        def _body(t, carry):
            for b in range(2):
                j = 2 * t + b
                pltpu.make_async_copy(y_hbm.at[sv.at[j]], rbs[b], gs[b]).wait()
                pltpu.async_copy(rbs[b], acc.at[dv.at[j]], ss, add=True).wait()
                pltpu.async_copy(y_hbm.at[sv.at[j + 2]], rbs[b], gs[b])
            return carry

        lax.fori_loop(0, CW // 2 - 1, _body, 0)
        for b in range(2):
            j = CW - 2 + b
            pltpu.make_async_copy(y_hbm.at[sv.at[j]], rbs[b], gs[b]).wait()
            pltpu.async_copy(rbs[b], acc.at[dv.at[j]], ss, add=True).wait()
        plsc.subcore_barrier()

        for k in range(RPT // CK):
            pltpu.sync_copy(acc.at[pl.ds(base + k * CK, CK)], rb0)

            @pl.when(cid == 0)
            def _():
                pltpu.sync_copy(rb0, o0_hbm.at[pl.ds(base + k * CK, CK)])

            @pl.when(cid != 0)
            def _():
                pltpu.sync_copy(rb0, o1_hbm.at[pl.ds(base + k * CK, CK)])


def _kc_body(plo0_ref, plo1_ref, phi0_ref, phi1_ref, d0_ref, d1_ref,
             bg_ref, w2_ref, b2_ref, o_ref):
    deg = d0_ref[:, :1] + d1_ref[:, :1] + 1.0
    dis = lax.rsqrt(deg)
    hlo = jnp.maximum((plo0_ref[...] + plo1_ref[...]) * dis + bg_ref[:, :HW], 0.0)
    hhi = jnp.maximum((phi0_ref[...] + phi1_ref[...]) * dis + bg_ref[:, HW:], 0.0)
    o = jnp.dot(hlo, w2_ref[:HW, :], preferred_element_type=jnp.float32)
    o = o + jnp.dot(hhi, w2_ref[HW:, :], preferred_element_type=jnp.float32)
    o = o + b2_ref[...]
    col = lax.broadcasted_iota(jnp.int32, (RBO, D), 1)
    o_ref[...] = jnp.where(col == 0, jax.nn.sigmoid(o), o)


def _kc(plo0, plo1, phi0, phi1, deg0, deg1, bgr, W2, b2r):
    return pl.pallas_call(
        _kc_body,
        grid=(GRID_O,),
        in_specs=[
            pl.BlockSpec((RBO, HW), lambda i: (i, 0)),
            pl.BlockSpec((RBO, HW), lambda i: (i, 0)),
            pl.BlockSpec((RBO, HW), lambda i: (i, 0)),
            pl.BlockSpec((RBO, HW), lambda i: (i, 0)),
            pl.BlockSpec((RBO, 16), lambda i: (i, 0)),
            pl.BlockSpec((RBO, 16), lambda i: (i, 0)),
            pl.BlockSpec((1, D), lambda i: (0, 0)),
            pl.BlockSpec((D, D), lambda i: (0, 0)),
            pl.BlockSpec((1, D), lambda i: (0, 0)),
        ],
        out_specs=pl.BlockSpec((RBO, D), lambda i: (i, 0)),
        out_shape=jax.ShapeDtypeStruct((N, D), jnp.float32),
    )(plo0, plo1, phi0, phi1, deg0, deg1, bgr, W2, b2r)


@jax.jit
def kernel(z, W1, b1, Wg, bg, W2, b2, edge_index):
    z_p = jnp.pad(z, ((0, NP - N), (0, 0)))
    b1r = b1.reshape(1, D)
    bgr = bg.reshape(1, D)
    b2r = b2.reshape(1, D)

    npad = EP - E
    pad_idx = (N + (jnp.arange(npad, dtype=jnp.int32) % (NP - N))).astype(jnp.int32)
    s_r = jnp.concatenate([edge_index[0], pad_idx]).reshape(NW, CW, CK)
    d_r = jnp.concatenate([edge_index[1], pad_idx]).reshape(NW, CW, CK)

    onehot = jnp.zeros((CK, 16), jnp.float32).at[:, 0].set(1.0)
    zeros16 = jnp.zeros((RPT, 16), jnp.float32)
    zerosrow = jnp.zeros((CK, HW), jnp.float32)

    deg0, deg1 = _deg_kernel(d_r, onehot, zeros16)
    ylo, yhi = _ka(z_p, W1, b1r, Wg, deg0, deg1)
    p0lo, p1lo, p0hi, p1hi = _seg_kernel(ylo, yhi, s_r, d_r, zerosrow)
    return _kc(p0lo, p1lo, p0hi, p1hi, deg0, deg1, bgr, W2, b2r)


# speedup vs baseline: 33.7521x; 1.0782x over previous
import functools

import jax
import jax.numpy as jnp
from jax import lax
from jax.experimental import pallas as pl
from jax.experimental.pallas import tpu as pltpu
from jax.experimental.pallas import tpu_sc as plsc

N = 10000
D = 128
E = 320000

NC = 2
NS = 16
NW = NC * NS
CK = 128
CW = 80
EP = NW * CW * CK
NP = 10240
RPT = NP // NS
HW = 64
GRID = 8
RB = NP // GRID
GRID_O = 10
RBO = N // GRID_O

_mesh = plsc.VectorSubcoreMesh(
    core_axis_name="c", subcore_axis_name="s", num_cores=NC, num_subcores=NS
)
_sc_params = pltpu.CompilerParams(use_tc_tiling_on_sc=False)


@functools.partial(
    pl.kernel,
    out_type=[jax.ShapeDtypeStruct((NP, 16), jnp.float32),
              jax.ShapeDtypeStruct((NP, 16), jnp.float32)],
    mesh=_mesh,
    compiler_params=_sc_params,
    scratch_types=[
        pltpu.VMEM((CW, CK), jnp.int32),
        pltpu.VMEM((CK, 16), jnp.float32),
        pltpu.VMEM((RPT, 16), jnp.float32),
        pltpu.VMEM_SHARED((NP, 16), jnp.float32),
        pltpu.SemaphoreType.DMA,
    ],
)
def _deg_kernel(d_hbm, oh_hbm, z16_hbm, out0_hbm, out1_hbm, dv, oh, zb, acc, sem):
    cid = lax.axis_index("c")
    sid = lax.axis_index("s")
    wid = sid * NC + cid
    base = sid * RPT
    pltpu.sync_copy(d_hbm.at[wid], dv)
    pltpu.sync_copy(oh_hbm, oh)
    pltpu.sync_copy(z16_hbm, zb)
    pltpu.sync_copy(zb, acc.at[pl.ds(base, RPT)])
    plsc.subcore_barrier()

    def _start(j, carry):
        pltpu.async_copy(oh, acc.at[dv.at[j]], sem, add=True)
        return carry

    lax.fori_loop(0, CW, _start, 0)

    def _drain(j, carry):
        pltpu.make_async_copy(oh, acc.at[dv.at[0]], sem).wait()
        return carry

    lax.fori_loop(0, CW, _drain, 0)
    plsc.subcore_barrier()
    pltpu.sync_copy(acc.at[pl.ds(base, RPT)], zb)

    @pl.when(cid == 0)
    def _():
        pltpu.sync_copy(zb, out0_hbm.at[pl.ds(base, RPT)])

    @pl.when(cid != 0)
    def _():
        pltpu.sync_copy(zb, out1_hbm.at[pl.ds(base, RPT)])


def _ka_body(z_ref, w1_ref, b1_ref, wg_ref, d0_ref, d1_ref, ylo_ref, yhi_ref):
    h = jnp.dot(z_ref[...], w1_ref[...], preferred_element_type=jnp.float32)
    h = jnp.maximum(h + b1_ref[...], 0.0)
    xw = jnp.dot(h, wg_ref[...], preferred_element_type=jnp.float32)
    deg = d0_ref[:, :1] + d1_ref[:, :1] + 1.0
    y = xw * lax.rsqrt(deg)
    ylo_ref[...] = y[:, :HW]
    yhi_ref[...] = y[:, HW:]


def _ka(z_p, W1, b1r, Wg, deg0, deg1):
    return pl.pallas_call(
        _ka_body,
        grid=(GRID,),
        in_specs=[
            pl.BlockSpec((RB, D), lambda i: (i, 0)),
            pl.BlockSpec((D, D), lambda i: (0, 0)),
            pl.BlockSpec((1, D), lambda i: (0, 0)),
            pl.BlockSpec((D, D), lambda i: (0, 0)),
            pl.BlockSpec((RB, 16), lambda i: (i, 0)),
            pl.BlockSpec((RB, 16), lambda i: (i, 0)),
        ],
        out_specs=[pl.BlockSpec((RB, HW), lambda i: (i, 0)),
                   pl.BlockSpec((RB, HW), lambda i: (i, 0))],
        out_shape=[jax.ShapeDtypeStruct((NP, HW), jnp.float32),
                   jax.ShapeDtypeStruct((NP, HW), jnp.float32)],
    )(z_p, W1, b1r, Wg, deg0, deg1)


@functools.partial(
    pl.kernel,
    out_type=[jax.ShapeDtypeStruct((NP, HW), jnp.float32)] * 4,
    mesh=_mesh,
    compiler_params=_sc_params,
    scratch_types=[
        pltpu.VMEM((CW, CK), jnp.int32),
        pltpu.VMEM((CW, CK), jnp.int32),
        pltpu.VMEM((CK, HW), jnp.float32),
        pltpu.VMEM((CK, HW), jnp.float32),
        pltpu.VMEM((CK, HW), jnp.float32),
        pltpu.VMEM((CK, HW), jnp.float32),
        pltpu.VMEM_SHARED((NP, HW), jnp.float32),
        pltpu.SemaphoreType.DMA,
        pltpu.SemaphoreType.DMA,
        pltpu.SemaphoreType.DMA,
        pltpu.SemaphoreType.DMA,
        pltpu.SemaphoreType.DMA,
        pltpu.SemaphoreType.DMA,
        pltpu.SemaphoreType.DMA,
        pltpu.SemaphoreType.DMA,
    ],
)
def _seg_kernel(ylo_hbm, yhi_hbm, s_hbm, d_hbm, zslab_hbm,
                p0lo_hbm, p1lo_hbm, p0hi_hbm, p1hi_hbm,
                sv, dv, rb0, rb1, rb2, rb3, acc,
                g0, g1, g2, g3, s0, s1, s2, s3):
    cid = lax.axis_index("c")
    sid = lax.axis_index("s")
    wid = sid * NC + cid
    base = sid * RPT
    pltpu.sync_copy(s_hbm.at[wid], sv)
    pltpu.sync_copy(d_hbm.at[wid], dv)
    rbs = (rb0, rb1, rb2, rb3)
    gs = (g0, g1, g2, g3)
    sse = (s0, s1, s2, s3)

    for y_hbm, o0_hbm, o1_hbm in ((ylo_hbm, p0lo_hbm, p1lo_hbm),
                                  (yhi_hbm, p0hi_hbm, p1hi_hbm)):
        @pl.when(cid == 0)
        def _():
            pltpu.sync_copy(y_hbm.at[pl.ds(base, RPT)], acc.at[pl.ds(base, RPT)])

        @pl.when(cid != 0)
        def _():
            pltpu.sync_copy(zslab_hbm, acc.at[pl.ds(base, RPT)])

        plsc.subcore_barrier()

        pltpu.async_copy(y_hbm.at[sv.at[0]], rbs[0], gs[0])
        pltpu.async_copy(y_hbm.at[sv.at[1]], rbs[1], gs[1])
        for j in (0, 1):
            pltpu.make_async_copy(y_hbm.at[sv.at[j]], rbs[j], gs[j]).wait()
            pltpu.async_copy(rbs[j], acc.at[dv.at[j]], sse[j], add=True)
            pltpu.async_copy(y_hbm.at[sv.at[j + 2]], rbs[j + 2], gs[j + 2])

        def _body(t, carry):
            for b4 in range(4):
                j = 2 + 4 * t + b4
                bb = (2 + b4) % 4
                br = (bb + 2) % 4
                pltpu.make_async_copy(y_hbm.at[sv.at[j]], rbs[bb], gs[bb]).wait()
                pltpu.async_copy(rbs[bb], acc.at[dv.at[j]], sse[bb], add=True)
                pltpu.make_async_copy(rbs[br], acc.at[dv.at[0]], sse[br]).wait()
                pltpu.async_copy(y_hbm.at[sv.at[j + 2]], rbs[br], gs[br])
            return carry

        lax.fori_loop(0, (CW - 4) // 4, _body, 0)
        for j in (CW - 2, CW - 1):
            b = j % 4
            pltpu.make_async_copy(y_hbm.at[sv.at[j]], rbs[b], gs[b]).wait()
            pltpu.async_copy(rbs[b], acc.at[dv.at[j]], sse[b], add=True)
        for b in range(4):
            pltpu.make_async_copy(rbs[b], acc.at[dv.at[0]], sse[b]).wait()
        plsc.subcore_barrier()

        @pl.when(cid == 0)
        def _():
            pltpu.sync_copy(acc.at[pl.ds(base, RPT)], o0_hbm.at[pl.ds(base, RPT)])

        @pl.when(cid != 0)
        def _():
            pltpu.sync_copy(acc.at[pl.ds(base, RPT)], o1_hbm.at[pl.ds(base, RPT)])


def _kc_body(plo0_ref, plo1_ref, phi0_ref, phi1_ref, d0_ref, d1_ref,
             bg_ref, w2_ref, b2_ref, o_ref):
    deg = d0_ref[:, :1] + d1_ref[:, :1] + 1.0
    dis = lax.rsqrt(deg)
    hlo = jnp.maximum((plo0_ref[...] + plo1_ref[...]) * dis + bg_ref[:, :HW], 0.0)
    hhi = jnp.maximum((phi0_ref[...] + phi1_ref[...]) * dis + bg_ref[:, HW:], 0.0)
    o = jnp.dot(hlo, w2_ref[:HW, :], preferred_element_type=jnp.float32)
    o = o + jnp.dot(hhi, w2_ref[HW:, :], preferred_element_type=jnp.float32)
    o = o + b2_ref[...]
    col = lax.broadcasted_iota(jnp.int32, (RBO, D), 1)
    o_ref[...] = jnp.where(col == 0, jax.nn.sigmoid(o), o)


def _kc(plo0, plo1, phi0, phi1, deg0, deg1, bgr, W2, b2r):
    return pl.pallas_call(
        _kc_body,
        grid=(GRID_O,),
        in_specs=[
            pl.BlockSpec((RBO, HW), lambda i: (i, 0)),
            pl.BlockSpec((RBO, HW), lambda i: (i, 0)),
            pl.BlockSpec((RBO, HW), lambda i: (i, 0)),
            pl.BlockSpec((RBO, HW), lambda i: (i, 0)),
            pl.BlockSpec((RBO, 16), lambda i: (i, 0)),
            pl.BlockSpec((RBO, 16), lambda i: (i, 0)),
            pl.BlockSpec((1, D), lambda i: (0, 0)),
            pl.BlockSpec((D, D), lambda i: (0, 0)),
            pl.BlockSpec((1, D), lambda i: (0, 0)),
        ],
        out_specs=pl.BlockSpec((RBO, D), lambda i: (i, 0)),
        out_shape=jax.ShapeDtypeStruct((N, D), jnp.float32),
    )(plo0, plo1, phi0, phi1, deg0, deg1, bgr, W2, b2r)


@jax.jit
def kernel(z, W1, b1, Wg, bg, W2, b2, edge_index):
    z_p = jnp.pad(z, ((0, NP - N), (0, 0)))
    b1r = b1.reshape(1, D)
    bgr = bg.reshape(1, D)
    b2r = b2.reshape(1, D)

    npad = EP - E
    pad_idx = (N + (jnp.arange(npad, dtype=jnp.int32) % (NP - N))).astype(jnp.int32)
    s_r = jnp.concatenate([edge_index[0], pad_idx]).reshape(NW, CW, CK)
    d_r = jnp.concatenate([edge_index[1], pad_idx]).reshape(NW, CW, CK)

    onehot = jnp.zeros((CK, 16), jnp.float32).at[:, 0].set(1.0)
    zeros16 = jnp.zeros((RPT, 16), jnp.float32)
    zslab = jnp.zeros((RPT, HW), jnp.float32)

    deg0, deg1 = _deg_kernel(d_r, onehot, zeros16)
    ylo, yhi = _ka(z_p, W1, b1r, Wg, deg0, deg1)
    p0lo, p1lo, p0hi, p1hi = _seg_kernel(ylo, yhi, s_r, d_r, zslab)
    return _kc(p0lo, p1lo, p0hi, p1hi, deg0, deg1, bgr, W2, b2r)


# speedup vs baseline: 36.4707x; 1.0805x over previous
import functools

import jax
import jax.numpy as jnp
from jax import lax
from jax.experimental import pallas as pl
from jax.experimental.pallas import tpu as pltpu
from jax.experimental.pallas import tpu_sc as plsc

N = 10000
D = 128
E = 320000

NC = 2
NS = 16
NW = NC * NS
CK = 128
CW = 80
EP = NW * CW * CK
NP = 10240
RPT = NP // NS
HW = 64
GRID = 8
RB = NP // GRID
GRID_O = 10
RBO = N // GRID_O

_mesh = plsc.VectorSubcoreMesh(
    core_axis_name="c", subcore_axis_name="s", num_cores=NC, num_subcores=NS
)
_sc_params = pltpu.CompilerParams(use_tc_tiling_on_sc=False)


@functools.partial(
    pl.kernel,
    out_type=[jax.ShapeDtypeStruct((NP, 16), jnp.float32),
              jax.ShapeDtypeStruct((NP, 16), jnp.float32)],
    mesh=_mesh,
    compiler_params=_sc_params,
    scratch_types=[
        pltpu.VMEM((CW, CK), jnp.int32),
        pltpu.VMEM((CK, 16), jnp.float32),
        pltpu.VMEM((RPT, 16), jnp.float32),
        pltpu.VMEM_SHARED((NP, 16), jnp.float32),
        pltpu.SemaphoreType.DMA,
    ],
)
def _deg_kernel(d_hbm, oh_hbm, z16_hbm, out0_hbm, out1_hbm, dv, oh, zb, acc, sem):
    cid = lax.axis_index("c")
    sid = lax.axis_index("s")
    wid = sid * NC + cid
    base = sid * RPT
    pltpu.sync_copy(d_hbm.at[wid], dv)
    pltpu.sync_copy(oh_hbm, oh)
    pltpu.sync_copy(z16_hbm, zb)
    pltpu.sync_copy(zb, acc.at[pl.ds(base, RPT)])
    plsc.subcore_barrier()

    def _start(j, carry):
        pltpu.async_copy(oh, acc.at[dv.at[j]], sem, add=True)
        return carry

    lax.fori_loop(0, CW, _start, 0)

    def _drain(j, carry):
        pltpu.make_async_copy(oh, acc.at[dv.at[0]], sem).wait()
        return carry

    lax.fori_loop(0, CW, _drain, 0)
    plsc.subcore_barrier()
    pltpu.sync_copy(acc.at[pl.ds(base, RPT)], zb)

    @pl.when(cid == 0)
    def _():
        pltpu.sync_copy(zb, out0_hbm.at[pl.ds(base, RPT)])

    @pl.when(cid != 0)
    def _():
        pltpu.sync_copy(zb, out1_hbm.at[pl.ds(base, RPT)])


def _ka_body(z_ref, w1_ref, b1_ref, wg_ref, d0_ref, d1_ref, y_ref):
    h = jnp.dot(z_ref[...], w1_ref[...], preferred_element_type=jnp.float32)
    h = jnp.maximum(h + b1_ref[...], 0.0)
    xw = jnp.dot(h, wg_ref[...], preferred_element_type=jnp.float32)
    deg = d0_ref[:, :1] + d1_ref[:, :1] + 1.0
    y_ref[...] = xw * lax.rsqrt(deg)


def _ka(z_p, W1, b1r, Wg, deg0, deg1):
    return pl.pallas_call(
        _ka_body,
        grid=(GRID,),
        in_specs=[
            pl.BlockSpec((RB, D), lambda i: (i, 0)),
            pl.BlockSpec((D, D), lambda i: (0, 0)),
            pl.BlockSpec((1, D), lambda i: (0, 0)),
            pl.BlockSpec((D, D), lambda i: (0, 0)),
            pl.BlockSpec((RB, 16), lambda i: (i, 0)),
            pl.BlockSpec((RB, 16), lambda i: (i, 0)),
        ],
        out_specs=pl.BlockSpec((RB, D), lambda i: (i, 0)),
        out_shape=jax.ShapeDtypeStruct((NP, D), jnp.float32),
    )(z_p, W1, b1r, Wg, deg0, deg1)


@functools.partial(
    pl.kernel,
    out_type=[jax.ShapeDtypeStruct((NP, D), jnp.float32)] * 2,
    mesh=_mesh,
    compiler_params=_sc_params,
    scratch_types=[
        pltpu.VMEM((CW, CK), jnp.int32),
        pltpu.VMEM((CW, CK), jnp.int32),
        pltpu.VMEM((CW, CK), jnp.int32),
        pltpu.VMEM((CK, HW), jnp.float32),
        pltpu.VMEM((CK, HW), jnp.float32),
        pltpu.VMEM((CK, HW), jnp.float32),
        pltpu.VMEM((CK, HW), jnp.float32),
        pltpu.VMEM_SHARED((NP, HW), jnp.float32),
        pltpu.SemaphoreType.DMA,
        pltpu.SemaphoreType.DMA,
        pltpu.SemaphoreType.DMA,
        pltpu.SemaphoreType.DMA,
        pltpu.SemaphoreType.DMA,
        pltpu.SemaphoreType.DMA,
        pltpu.SemaphoreType.DMA,
        pltpu.SemaphoreType.DMA,
    ],
)
def _seg_kernel(y2_hbm, slo_hbm, shi_hbm, d_hbm, zslab_hbm,
                p0_hbm, p1_hbm,
                svlo, svhi, dv, rb0, rb1, rb2, rb3, acc,
                g0, g1, g2, g3, s0, s1, s2, s3):
    cid = lax.axis_index("c")
    sid = lax.axis_index("s")
    wid = sid * NC + cid
    base = sid * RPT
    pltpu.sync_copy(slo_hbm.at[wid], svlo)
    pltpu.sync_copy(shi_hbm.at[wid], svhi)
    pltpu.sync_copy(d_hbm.at[wid], dv)
    rbs = (rb0, rb1, rb2, rb3)
    gs = (g0, g1, g2, g3)
    sse = (s0, s1, s2, s3)

    for sv, off in ((svlo, 0), (svhi, HW)):
        pltpu.sync_copy(zslab_hbm, acc.at[pl.ds(base, RPT)])
        plsc.subcore_barrier()

        pltpu.async_copy(y2_hbm.at[sv.at[0]], rbs[0], gs[0])
        pltpu.async_copy(y2_hbm.at[sv.at[1]], rbs[1], gs[1])
        for j in (0, 1):
            pltpu.make_async_copy(y2_hbm.at[sv.at[j]], rbs[j], gs[j]).wait()
            pltpu.async_copy(rbs[j], acc.at[dv.at[j]], sse[j], add=True)
            pltpu.async_copy(y2_hbm.at[sv.at[j + 2]], rbs[j + 2], gs[j + 2])

        def _body(t, carry):
            for b4 in range(4):
                j = 2 + 4 * t + b4
                bb = (2 + b4) % 4
                br = (bb + 2) % 4
                pltpu.make_async_copy(y2_hbm.at[sv.at[j]], rbs[bb], gs[bb]).wait()
                pltpu.async_copy(rbs[bb], acc.at[dv.at[j]], sse[bb], add=True)
                pltpu.make_async_copy(rbs[br], acc.at[dv.at[0]], sse[br]).wait()
                pltpu.async_copy(y2_hbm.at[sv.at[j + 2]], rbs[br], gs[br])
            return carry

        lax.fori_loop(0, (CW - 4) // 4, _body, 0)
        for j in (CW - 2, CW - 1):
            b = j % 4
            pltpu.make_async_copy(y2_hbm.at[sv.at[j]], rbs[b], gs[b]).wait()
            pltpu.async_copy(rbs[b], acc.at[dv.at[j]], sse[b], add=True)
        for b in range(4):
            pltpu.make_async_copy(rbs[b], acc.at[dv.at[0]], sse[b]).wait()
        plsc.subcore_barrier()

        @pl.when(cid == 0)
        def _():
            pltpu.sync_copy(acc.at[pl.ds(base, RPT)],
                            p0_hbm.at[pl.ds(base, RPT), pl.ds(off, HW)])

        @pl.when(cid != 0)
        def _():
            pltpu.sync_copy(acc.at[pl.ds(base, RPT)],
                            p1_hbm.at[pl.ds(base, RPT), pl.ds(off, HW)])


def _kc_body(p0_ref, p1_ref, y_ref, d0_ref, d1_ref, bg_ref, w2_ref, b2_ref, o_ref):
    deg = d0_ref[:, :1] + d1_ref[:, :1] + 1.0
    dis = lax.rsqrt(deg)
    h = jnp.maximum((p0_ref[...] + p1_ref[...] + y_ref[...]) * dis + bg_ref[...], 0.0)
    o = jnp.dot(h, w2_ref[...], preferred_element_type=jnp.float32)
    o = o + b2_ref[...]
    col = lax.broadcasted_iota(jnp.int32, (RB, D), 1)
    o_ref[...] = jnp.where(col == 0, jax.nn.sigmoid(o), o)


def _kc(p0, p1, y, deg0, deg1, bgr, W2, b2r):
    return pl.pallas_call(
        _kc_body,
        grid=(GRID,),
        in_specs=[
            pl.BlockSpec((RB, D), lambda i: (i, 0)),
            pl.BlockSpec((RB, D), lambda i: (i, 0)),
            pl.BlockSpec((RB, D), lambda i: (i, 0)),
            pl.BlockSpec((RB, 16), lambda i: (i, 0)),
            pl.BlockSpec((RB, 16), lambda i: (i, 0)),
            pl.BlockSpec((1, D), lambda i: (0, 0)),
            pl.BlockSpec((D, D), lambda i: (0, 0)),
            pl.BlockSpec((1, D), lambda i: (0, 0)),
        ],
        out_specs=pl.BlockSpec((RB, D), lambda i: (i, 0)),
        out_shape=jax.ShapeDtypeStruct((NP, D), jnp.float32),
    )(p0, p1, y, deg0, deg1, bgr, W2, b2r)


@jax.jit
def kernel(z, W1, b1, Wg, bg, W2, b2, edge_index):
    z_p = jnp.pad(z, ((0, NP - N), (0, 0)))
    b1r = b1.reshape(1, D)
    bgr = bg.reshape(1, D)
    b2r = b2.reshape(1, D)

    npad = EP - E
    pad_idx = (N + (jnp.arange(npad, dtype=jnp.int32) % (NP - N))).astype(jnp.int32)
    s_full = jnp.concatenate([edge_index[0], pad_idx])
    slo_r = (s_full * 2).reshape(NW, CW, CK)
    shi_r = (s_full * 2 + 1).reshape(NW, CW, CK)
    d_r = jnp.concatenate([edge_index[1], pad_idx]).reshape(NW, CW, CK)

    onehot = jnp.zeros((CK, 16), jnp.float32).at[:, 0].set(1.0)
    zeros16 = jnp.zeros((RPT, 16), jnp.float32)
    zslab = jnp.zeros((RPT, HW), jnp.float32)

    deg0, deg1 = _deg_kernel(d_r, onehot, zeros16)
    y = _ka(z_p, W1, b1r, Wg, deg0, deg1)
    y2 = y.reshape(2 * NP, HW)
    p0, p1 = _seg_kernel(y2, slo_r, shi_r, d_r, zslab)
    out = _kc(p0, p1, y, deg0, deg1, bgr, W2, b2r)
    return out[:N]


# speedup vs baseline: 39.2952x; 1.0774x over previous
import functools

import jax
import jax.numpy as jnp
from jax import lax
from jax.experimental import pallas as pl
from jax.experimental.pallas import tpu as pltpu
from jax.experimental.pallas import tpu_sc as plsc

N = 10000
D = 128
E = 320000

NC = 2
NS = 16
NW = NC * NS
CK = 128
CW = 80
EP = NW * CW * CK
NP = 10240
RPT = NP // NS
HW = 64
GRID = 8
RB = NP // GRID
GRID_O = 10
RBO = N // GRID_O
GRID_O = 10
RBO = N // GRID_O

_mesh = plsc.VectorSubcoreMesh(
    core_axis_name="c", subcore_axis_name="s", num_cores=NC, num_subcores=NS
)
_sc_params = pltpu.CompilerParams(use_tc_tiling_on_sc=False)


@functools.partial(
    pl.kernel,
    out_type=[jax.ShapeDtypeStruct((NP, 16), jnp.float32),
              jax.ShapeDtypeStruct((NP, 16), jnp.float32)],
    mesh=_mesh,
    compiler_params=_sc_params,
    scratch_types=[
        pltpu.VMEM((CW, CK), jnp.int32),
        pltpu.VMEM((CK, 16), jnp.float32),
        pltpu.VMEM((RPT, 16), jnp.float32),
        pltpu.VMEM_SHARED((NP, 16), jnp.float32),
        pltpu.SemaphoreType.DMA,
    ],
)
def _deg_kernel(d_hbm, oh_hbm, z16_hbm, out0_hbm, out1_hbm, dv, oh, zb, acc, sem):
    cid = lax.axis_index("c")
    sid = lax.axis_index("s")
    wid = sid * NC + cid
    base = sid * RPT
    pltpu.sync_copy(d_hbm.at[wid], dv)
    pltpu.sync_copy(oh_hbm, oh)
    pltpu.sync_copy(z16_hbm, zb)
    pltpu.sync_copy(zb, acc.at[pl.ds(base, RPT)])
    plsc.subcore_barrier()

    def _start(j, carry):
        pltpu.async_copy(oh, acc.at[dv.at[j]], sem, add=True)
        return carry

    lax.fori_loop(0, CW, _start, 0)

    def _drain(j, carry):
        pltpu.make_async_copy(oh, acc.at[dv.at[0]], sem).wait()
        return carry

    lax.fori_loop(0, CW, _drain, 0)
    plsc.subcore_barrier()
    pltpu.sync_copy(acc.at[pl.ds(base, RPT)], zb)

    @pl.when(cid == 0)
    def _():
        pltpu.sync_copy(zb, out0_hbm.at[pl.ds(base, RPT)])

    @pl.when(cid != 0)
    def _():
        pltpu.sync_copy(zb, out1_hbm.at[pl.ds(base, RPT)])


def _ka_body(z_ref, w1_ref, b1_ref, wg_ref, d0_ref, d1_ref, y_ref):
    h = jnp.dot(z_ref[...], w1_ref[...], preferred_element_type=jnp.float32)
    h = jnp.maximum(h + b1_ref[...], 0.0)
    xw = jnp.dot(h, wg_ref[...], preferred_element_type=jnp.float32)
    deg = d0_ref[:, :1] + d1_ref[:, :1] + 1.0
    y_ref[...] = xw * lax.rsqrt(deg)


def _ka(z_p, W1, b1r, Wg, deg0, deg1):
    return pl.pallas_call(
        _ka_body,
        grid=(GRID,),
        in_specs=[
            pl.BlockSpec((RB, D), lambda i: (i, 0)),
            pl.BlockSpec((D, D), lambda i: (0, 0)),
            pl.BlockSpec((1, D), lambda i: (0, 0)),
            pl.BlockSpec((D, D), lambda i: (0, 0)),
            pl.BlockSpec((RB, 16), lambda i: (i, 0)),
            pl.BlockSpec((RB, 16), lambda i: (i, 0)),
        ],
        out_specs=pl.BlockSpec((RB, D), lambda i: (i, 0)),
        out_shape=jax.ShapeDtypeStruct((NP, D), jnp.float32),
    )(z_p, W1, b1r, Wg, deg0, deg1)


@functools.partial(
    pl.kernel,
    out_type=[jax.ShapeDtypeStruct((NP, D), jnp.float32)] * 2,
    mesh=_mesh,
    compiler_params=_sc_params,
    scratch_types=[
        pltpu.VMEM((CW, CK), jnp.int32),
        pltpu.VMEM((CW, CK), jnp.int32),
        pltpu.VMEM((CW, CK), jnp.int32),
        pltpu.VMEM((CW, CK), jnp.int32),
        pltpu.VMEM((CK, HW), jnp.float32),
        pltpu.VMEM((CK, HW), jnp.float32),
        pltpu.VMEM((CK, HW), jnp.float32),
        pltpu.VMEM((CK, HW), jnp.float32),
        pltpu.VMEM((CK, HW), jnp.float32),
        pltpu.VMEM((CK, HW), jnp.float32),
        pltpu.VMEM_SHARED((NP, HW), jnp.float32),
        pltpu.SemaphoreType.DMA,
        pltpu.SemaphoreType.DMA,
        pltpu.SemaphoreType.DMA,
        pltpu.SemaphoreType.DMA,
        pltpu.SemaphoreType.DMA,
        pltpu.SemaphoreType.DMA,
        pltpu.SemaphoreType.DMA,
        pltpu.SemaphoreType.DMA,
        pltpu.SemaphoreType.DMA,
        pltpu.SemaphoreType.DMA,
        pltpu.SemaphoreType.DMA,
        pltpu.SemaphoreType.DMA,
    ],
)
def _seg_kernel(y2_hbm, s_hbm, d_hbm, zslab_hbm, p0_hbm, p1_hbm,
                svo, svlo, svhi, dv, rb0, rb1, rb2, rb3, rb4, rb5, acc,
                g0, g1, g2, g3, g4, g5, s0, s1, s2, s3, s4, s5):
    cid = lax.axis_index("c")
    sid = lax.axis_index("s")
    wid = sid * NC + cid
    base = sid * RPT
    pltpu.sync_copy(s_hbm.at[wid], svo)
    pltpu.sync_copy(d_hbm.at[wid], dv)

    def _mkidx(r, carry):
        for c8 in range(CK // 16):
            v2 = svo[r, pl.ds(16 * c8, 16)] * 2
            svlo[r, pl.ds(16 * c8, 16)] = v2
            svhi[r, pl.ds(16 * c8, 16)] = v2 + 1
        return carry

    lax.fori_loop(0, CW, _mkidx, 0)

    rbs = (rb0, rb1, rb2, rb3, rb4, rb5)
    gs = (g0, g1, g2, g3, g4, g5)
    sse = (s0, s1, s2, s3, s4, s5)

    for sv, off in ((svlo, 0), (svhi, HW)):
        pltpu.sync_copy(zslab_hbm, acc.at[pl.ds(base, RPT)])
        plsc.subcore_barrier()

        for b in range(3):
            pltpu.async_copy(y2_hbm.at[sv.at[b]], rbs[b], gs[b])
        for j in range(3):
            pltpu.make_async_copy(y2_hbm.at[sv.at[j]], rbs[j], gs[j]).wait()
            pltpu.async_copy(rbs[j], acc.at[dv.at[j]], sse[j], add=True)
            pltpu.async_copy(y2_hbm.at[sv.at[j + 3]], rbs[j + 3], gs[j + 3])

        def _body(t, carry):
            for b6 in range(6):
                j = 3 + 6 * t + b6
                bb = (3 + b6) % 6
                br = b6
                pltpu.make_async_copy(y2_hbm.at[sv.at[j]], rbs[bb], gs[bb]).wait()
                pltpu.async_copy(rbs[bb], acc.at[dv.at[j]], sse[bb], add=True)
                pltpu.make_async_copy(rbs[br], acc.at[dv.at[0]], sse[br]).wait()
                pltpu.async_copy(y2_hbm.at[sv.at[j + 3]], rbs[br], gs[br])
            return carry

        lax.fori_loop(0, (CW - 8) // 6, _body, 0)
        for j in (CW - 5, CW - 4):
            bb = j % 6
            br = (j + 3) % 6
            pltpu.make_async_copy(y2_hbm.at[sv.at[j]], rbs[bb], gs[bb]).wait()
            pltpu.async_copy(rbs[bb], acc.at[dv.at[j]], sse[bb], add=True)
            pltpu.make_async_copy(rbs[br], acc.at[dv.at[0]], sse[br]).wait()
            pltpu.async_copy(y2_hbm.at[sv.at[j + 3]], rbs[br], gs[br])
        for j in (CW - 3, CW - 2, CW - 1):
            bb = j % 6
            pltpu.make_async_copy(y2_hbm.at[sv.at[j]], rbs[bb], gs[bb]).wait()
            pltpu.async_copy(rbs[bb], acc.at[dv.at[j]], sse[bb], add=True)
        for b in range(6):
            pltpu.make_async_copy(rbs[b], acc.at[dv.at[0]], sse[b]).wait()
        plsc.subcore_barrier()

        @pl.when(cid == 0)
        def _():
            pltpu.sync_copy(acc.at[pl.ds(base, RPT)],
                            p0_hbm.at[pl.ds(base, RPT), pl.ds(off, HW)])

        @pl.when(cid != 0)
        def _():
            pltpu.sync_copy(acc.at[pl.ds(base, RPT)],
                            p1_hbm.at[pl.ds(base, RPT), pl.ds(off, HW)])


def _kc_body(p0_ref, p1_ref, y_ref, d0_ref, d1_ref, bg_ref, w2_ref, b2_ref, o_ref):
    deg = d0_ref[:, :1] + d1_ref[:, :1] + 1.0
    dis = lax.rsqrt(deg)
    h = jnp.maximum((p0_ref[...] + p1_ref[...] + y_ref[...]) * dis + bg_ref[...], 0.0)
    o = jnp.dot(h, w2_ref[...], preferred_element_type=jnp.float32)
    o = o + b2_ref[...]
    col = lax.broadcasted_iota(jnp.int32, (RBO, D), 1)
    o_ref[...] = jnp.where(col == 0, jax.nn.sigmoid(o), o)


def _kc(p0, p1, y, deg0, deg1, bgr, W2, b2r):
    return pl.pallas_call(
        _kc_body,
        grid=(GRID_O,),
        in_specs=[
            pl.BlockSpec((RBO, D), lambda i: (i, 0)),
            pl.BlockSpec((RBO, D), lambda i: (i, 0)),
            pl.BlockSpec((RBO, D), lambda i: (i, 0)),
            pl.BlockSpec((RBO, 16), lambda i: (i, 0)),
            pl.BlockSpec((RBO, 16), lambda i: (i, 0)),
            pl.BlockSpec((1, D), lambda i: (0, 0)),
            pl.BlockSpec((D, D), lambda i: (0, 0)),
            pl.BlockSpec((1, D), lambda i: (0, 0)),
        ],
        out_specs=pl.BlockSpec((RBO, D), lambda i: (i, 0)),
        out_shape=jax.ShapeDtypeStruct((N, D), jnp.float32),
    )(p0, p1, y, deg0, deg1, bgr, W2, b2r)


@jax.jit
def kernel(z, W1, b1, Wg, bg, W2, b2, edge_index):
    z_p = jnp.pad(z, ((0, NP - N), (0, 0)))
    b1r = b1.reshape(1, D)
    bgr = bg.reshape(1, D)
    b2r = b2.reshape(1, D)

    npad = EP - E
    pad_idx = (N + (jnp.arange(npad, dtype=jnp.int32) % (NP - N))).astype(jnp.int32)
    s_r = jnp.concatenate([edge_index[0], pad_idx]).reshape(NW, CW, CK)
    d_r = jnp.concatenate([edge_index[1], pad_idx]).reshape(NW, CW, CK)

    onehot = jnp.zeros((CK, 16), jnp.float32).at[:, 0].set(1.0)
    zeros16 = jnp.zeros((RPT, 16), jnp.float32)
    zslab = jnp.zeros((RPT, HW), jnp.float32)

    deg0, deg1 = _deg_kernel(d_r, onehot, zeros16)
    y = _ka(z_p, W1, b1r, Wg, deg0, deg1)
    y2 = y.reshape(2 * NP, HW)
    p0, p1 = _seg_kernel(y2, s_r, d_r, zslab)
    return _kc(p0, p1, y, deg0, deg1, bgr, W2, b2r)


# speedup vs baseline: 42.0083x; 1.0690x over previous
import functools

import jax
import jax.numpy as jnp
from jax import lax
from jax.experimental import pallas as pl
from jax.experimental.pallas import tpu as pltpu
from jax.experimental.pallas import tpu_sc as plsc

N = 10000
D = 128
E = 320000

NC = 2
NS = 16
NW = NC * NS
CK = 128
CW = 80
EP = NW * CW * CK
NP = 10240
RPT = NP // NS
HW = 64
GRID = 8
RB = NP // GRID
GRID_O = 10
RBO = 1024

_mesh = plsc.VectorSubcoreMesh(
    core_axis_name="c", subcore_axis_name="s", num_cores=NC, num_subcores=NS
)
_sc_params = pltpu.CompilerParams(use_tc_tiling_on_sc=False)


@functools.partial(
    pl.kernel,
    out_type=[jax.ShapeDtypeStruct((NP, 16), jnp.float32),
              jax.ShapeDtypeStruct((NP, 16), jnp.float32)],
    mesh=_mesh,
    compiler_params=_sc_params,
    scratch_types=[
        pltpu.VMEM((CW, CK), jnp.int32),
        pltpu.VMEM((CK, 16), jnp.float32),
        pltpu.VMEM((RPT, 16), jnp.float32),
        pltpu.VMEM_SHARED((NP, 16), jnp.float32),
        pltpu.SemaphoreType.DMA,
    ],
)
def _deg_kernel(d_hbm, oh_hbm, z16_hbm, out0_hbm, out1_hbm, dv, oh, zb, acc, sem):
    cid = lax.axis_index("c")
    sid = lax.axis_index("s")
    wid = sid * NC + cid
    base = sid * RPT
    pltpu.sync_copy(d_hbm.at[wid], dv)
    pltpu.sync_copy(oh_hbm, oh)
    pltpu.sync_copy(z16_hbm, zb)
    pltpu.sync_copy(zb, acc.at[pl.ds(base, RPT)])
    plsc.subcore_barrier()

    def _start(j, carry):
        pltpu.async_copy(oh, acc.at[dv.at[j]], sem, add=True)
        return carry

    lax.fori_loop(0, CW, _start, 0)

    def _drain(j, carry):
        pltpu.make_async_copy(oh, acc.at[dv.at[0]], sem).wait()
        return carry

    lax.fori_loop(0, CW, _drain, 0)
    plsc.subcore_barrier()
    pltpu.sync_copy(acc.at[pl.ds(base, RPT)], zb)

    @pl.when(cid == 0)
    def _():
        pltpu.sync_copy(zb, out0_hbm.at[pl.ds(base, RPT)])

    @pl.when(cid != 0)
    def _():
        pltpu.sync_copy(zb, out1_hbm.at[pl.ds(base, RPT)])


def _dis_from_views(d0v, d1v, nrows):
    nv = nrows // 8
    dv = d0v + d1v
    sel = (lax.broadcasted_iota(jnp.int32, (nrows, nv), 0) // 8
           == lax.broadcasted_iota(jnp.int32, (nrows, nv), 1)).astype(jnp.float32)
    rep = jnp.dot(sel, dv, preferred_element_type=jnp.float32)
    lane = (lax.broadcasted_iota(jnp.int32, (nrows, 128), 1)
            == 16 * (lax.broadcasted_iota(jnp.int32, (nrows, 128), 0) % 8))
    deg = jnp.sum(jnp.where(lane, rep, 0.0), axis=1, keepdims=True) + 1.0
    return lax.rsqrt(deg)


def _ka_body(z_ref, w1_ref, b1_ref, wg_ref, d0_ref, d1_ref, y_ref):
    h = jnp.dot(z_ref[...], w1_ref[...], preferred_element_type=jnp.float32)
    h = jnp.maximum(h + b1_ref[...], 0.0)
    xw = jnp.dot(h, wg_ref[...], preferred_element_type=jnp.float32)
    y_ref[...] = xw * _dis_from_views(d0_ref[...], d1_ref[...], RB)


def _ka(z_p, W1, b1r, Wg, deg0, deg1):
    return pl.pallas_call(
        _ka_body,
        grid=(GRID,),
        in_specs=[
            pl.BlockSpec((RB, D), lambda i: (i, 0)),
            pl.BlockSpec((D, D), lambda i: (0, 0)),
            pl.BlockSpec((1, D), lambda i: (0, 0)),
            pl.BlockSpec((D, D), lambda i: (0, 0)),
            pl.BlockSpec((RB // 8, D), lambda i: (i, 0)),
            pl.BlockSpec((RB // 8, D), lambda i: (i, 0)),
        ],
        out_specs=pl.BlockSpec((RB, D), lambda i: (i, 0)),
        out_shape=jax.ShapeDtypeStruct((NP, D), jnp.float32),
    )(z_p, W1, b1r, Wg, deg0, deg1)


@functools.partial(
    pl.kernel,
    out_type=[jax.ShapeDtypeStruct((NP, D), jnp.float32)] * 2,
    mesh=_mesh,
    compiler_params=_sc_params,
    scratch_types=[
        pltpu.VMEM((CW, CK), jnp.int32),
        pltpu.VMEM((CW, CK), jnp.int32),
        pltpu.VMEM((CK, HW), jnp.float32),
        pltpu.VMEM((CK, HW), jnp.float32),
        pltpu.VMEM((CK, HW), jnp.float32),
        pltpu.VMEM((CK, HW), jnp.float32),
        pltpu.VMEM((CK, HW), jnp.float32),
        pltpu.VMEM((CK, HW), jnp.float32),
        pltpu.VMEM((CK, HW), jnp.float32),
        pltpu.VMEM((CK, HW), jnp.float32),
        pltpu.VMEM_SHARED((NP, HW), jnp.float32),
        [pltpu.SemaphoreType.DMA] * 8,
        [pltpu.SemaphoreType.DMA] * 8,
    ],
)
def _seg_kernel(y2_hbm, s_hbm, d_hbm, zslab_hbm, p0_hbm, p1_hbm,
                sv, dv, rb0, rb1, rb2, rb3, rb4, rb5, rb6, rb7,
                acc, gs, sse):
    cid = lax.axis_index("c")
    sid = lax.axis_index("s")
    wid = sid * NC + cid
    base = sid * RPT
    pltpu.sync_copy(s_hbm.at[wid], sv)
    pltpu.sync_copy(d_hbm.at[wid], dv)

    def _mul2(r, carry):
        for c8 in range(CK // 16):
            sl = pl.ds(16 * c8, 16)
            sv[r, sl] = sv[r, sl] * 2
        return carry

    def _bump(r, carry):
        for c8 in range(CK // 16):
            sl = pl.ds(16 * c8, 16)
            sv[r, sl] = sv[r, sl] + 1
        return carry

    lax.fori_loop(0, CW, _mul2, 0)

    rbs = (rb0, rb1, rb2, rb3, rb4, rb5, rb6, rb7)
    NB = 8

    for phase, off in ((0, 0), (1, HW)):
        if phase == 1:
            lax.fori_loop(0, CW, _bump, 0)
        pltpu.sync_copy(zslab_hbm, acc.at[pl.ds(base, RPT)])
        plsc.subcore_barrier()

        for b in range(NB // 2):
            pltpu.async_copy(y2_hbm.at[sv.at[b]], rbs[b], gs[b])
        for j in range(NB // 2):
            pltpu.make_async_copy(y2_hbm.at[sv.at[j]], rbs[j], gs[j]).wait()
            pltpu.async_copy(rbs[j], acc.at[dv.at[j]], sse[j], add=True)
            pltpu.async_copy(y2_hbm.at[sv.at[j + 4]], rbs[j + 4], gs[j + 4])

        def _body(t, carry):
            for b8 in range(NB):
                j = 4 + NB * t + b8
                bb = (4 + b8) % NB
                br = b8
                pltpu.make_async_copy(y2_hbm.at[sv.at[j]], rbs[bb], gs[bb]).wait()
                pltpu.async_copy(rbs[bb], acc.at[dv.at[j]], sse[bb], add=True)
                pltpu.make_async_copy(rbs[br], acc.at[dv.at[0]], sse[br]).wait()
                pltpu.async_copy(y2_hbm.at[sv.at[j + 4]], rbs[br], gs[br])
            return carry

        lax.fori_loop(0, (CW - 8) // NB, _body, 0)
        for j in (CW - 4, CW - 3, CW - 2, CW - 1):
            bb = j % NB
            pltpu.make_async_copy(y2_hbm.at[sv.at[j]], rbs[bb], gs[bb]).wait()
            pltpu.async_copy(rbs[bb], acc.at[dv.at[j]], sse[bb], add=True)
        for b in range(NB):
            pltpu.make_async_copy(rbs[b], acc.at[dv.at[0]], sse[b]).wait()
        plsc.subcore_barrier()

        @pl.when(cid == 0)
        def _():
            pltpu.sync_copy(acc.at[pl.ds(base, RPT)],
                            p0_hbm.at[pl.ds(base, RPT), pl.ds(off, HW)])

        @pl.when(cid != 0)
        def _():
            pltpu.sync_copy(acc.at[pl.ds(base, RPT)],
                            p1_hbm.at[pl.ds(base, RPT), pl.ds(off, HW)])


def _kc_body(p0_ref, p1_ref, y_ref, d0_ref, d1_ref, bg_ref, w2_ref, b2_ref, o_ref):
    dis = _dis_from_views(d0_ref[...], d1_ref[...], RBO)
    h = jnp.maximum((p0_ref[...] + p1_ref[...] + y_ref[...]) * dis + bg_ref[...], 0.0)
    o = jnp.dot(h, w2_ref[...], preferred_element_type=jnp.float32)
    o = o + b2_ref[...]
    col = lax.broadcasted_iota(jnp.int32, (RBO, D), 1)
    o_ref[...] = jnp.where(col == 0, jax.nn.sigmoid(o), o)


def _kc(p0, p1, y, deg0, deg1, bgr, W2, b2r):
    return pl.pallas_call(
        _kc_body,
        grid=(GRID_O,),
        in_specs=[
            pl.BlockSpec((RBO, D), lambda i: (i, 0)),
            pl.BlockSpec((RBO, D), lambda i: (i, 0)),
            pl.BlockSpec((RBO, D), lambda i: (i, 0)),
            pl.BlockSpec((RBO // 8, D), lambda i: (i, 0)),
            pl.BlockSpec((RBO // 8, D), lambda i: (i, 0)),
            pl.BlockSpec((1, D), lambda i: (0, 0)),
            pl.BlockSpec((D, D), lambda i: (0, 0)),
            pl.BlockSpec((1, D), lambda i: (0, 0)),
        ],
        out_specs=pl.BlockSpec((RBO, D), lambda i: (i, 0)),
        out_shape=jax.ShapeDtypeStruct((N, D), jnp.float32),
    )(p0, p1, y, deg0, deg1, bgr, W2, b2r)


@jax.jit
def kernel(z, W1, b1, Wg, bg, W2, b2, edge_index):
    z_p = jnp.pad(z, ((0, NP - N), (0, 0)))
    b1r = b1.reshape(1, D)
    bgr = bg.reshape(1, D)
    b2r = b2.reshape(1, D)

    npad = EP - E
    pad_idx = (N + (jnp.arange(npad, dtype=jnp.int32) % (NP - N))).astype(jnp.int32)
    s_r = jnp.concatenate([edge_index[0], pad_idx]).reshape(NW, CW, CK)
    d_r = jnp.concatenate([edge_index[1], pad_idx]).reshape(NW, CW, CK)

    onehot = jnp.zeros((CK, 16), jnp.float32).at[:, 0].set(1.0)
    zeros16 = jnp.zeros((RPT, 16), jnp.float32)
    zslab = jnp.zeros((RPT, HW), jnp.float32)

    deg0, deg1 = _deg_kernel(d_r, onehot, zeros16)
    deg0v = deg0.reshape(NP // 8, D)
    deg1v = deg1.reshape(NP // 8, D)
    y = _ka(z_p, W1, b1r, Wg, deg0v, deg1v)
    y2 = y.reshape(2 * NP, HW)
    p0, p1 = _seg_kernel(y2, s_r, d_r, zslab)
    return _kc(p0, p1, y, deg0v, deg1v, bgr, W2, b2r)


# speedup vs baseline: 43.5853x; 1.0375x over previous
import functools

import jax
import jax.numpy as jnp
from jax import lax
from jax.experimental import pallas as pl
from jax.experimental.pallas import tpu as pltpu
from jax.experimental.pallas import tpu_sc as plsc

N = 10000
D = 128
E = 320000

NC = 2
NS = 16
NW = NC * NS
CK = 128
CW = 80
EP = NW * CW * CK
NP = 10240
RPT = NP // NS
HW = 64
GRID = 8
RB = NP // GRID
GRID_O = 10
RBO = 1024

_mesh = plsc.VectorSubcoreMesh(
    core_axis_name="c", subcore_axis_name="s", num_cores=NC, num_subcores=NS
)
_sc_params = pltpu.CompilerParams(use_tc_tiling_on_sc=False)


@functools.partial(
    pl.kernel,
    out_type=[jax.ShapeDtypeStruct((NP, 16), jnp.float32),
              jax.ShapeDtypeStruct((NP, 16), jnp.float32)],
    mesh=_mesh,
    compiler_params=_sc_params,
    scratch_types=[
        pltpu.VMEM((CW, CK), jnp.int32),
        pltpu.VMEM((CK, 16), jnp.float32),
        pltpu.VMEM((RPT, 16), jnp.float32),
        pltpu.VMEM_SHARED((NP, 16), jnp.float32),
        pltpu.SemaphoreType.DMA,
    ],
)
def _deg_kernel(d_hbm, oh_hbm, z16_hbm, out0_hbm, out1_hbm, dv, oh, zb, acc, sem):
    cid = lax.axis_index("c")
    sid = lax.axis_index("s")
    wid = sid * NC + cid
    base = sid * RPT
    pltpu.sync_copy(d_hbm.at[wid], dv)
    pltpu.sync_copy(oh_hbm, oh)
    pltpu.sync_copy(z16_hbm, zb)
    pltpu.sync_copy(zb, acc.at[pl.ds(base, RPT)])
    plsc.subcore_barrier()

    def _start(j, carry):
        pltpu.async_copy(oh, acc.at[dv.at[j]], sem, add=True)
        return carry

    lax.fori_loop(0, CW, _start, 0)

    def _drain(j, carry):
        pltpu.make_async_copy(oh, acc.at[dv.at[0]], sem).wait()
        return carry

    lax.fori_loop(0, CW, _drain, 0)
    plsc.subcore_barrier()
    pltpu.sync_copy(acc.at[pl.ds(base, RPT)], zb)

    @pl.when(cid == 0)
    def _():
        pltpu.sync_copy(zb, out0_hbm.at[pl.ds(base, RPT)])

    @pl.when(cid != 0)
    def _():
        pltpu.sync_copy(zb, out1_hbm.at[pl.ds(base, RPT)])


def _dis_from_views(d0v, d1v, nrows):
    nv = nrows // 8
    dv = d0v + d1v
    sel = (lax.broadcasted_iota(jnp.int32, (nrows, nv), 0) // 8
           == lax.broadcasted_iota(jnp.int32, (nrows, nv), 1)).astype(jnp.float32)
    rep = jnp.dot(sel, dv, preferred_element_type=jnp.float32)
    lane = (lax.broadcasted_iota(jnp.int32, (nrows, 128), 1)
            == 16 * (lax.broadcasted_iota(jnp.int32, (nrows, 128), 0) % 8))
    deg = jnp.sum(jnp.where(lane, rep, 0.0), axis=1, keepdims=True) + 1.0
    return lax.rsqrt(deg)


def _ka_body(z_ref, w1_ref, b1_ref, wg_ref, d0_ref, d1_ref, y_ref):
    h = jnp.dot(z_ref[...], w1_ref[...], preferred_element_type=jnp.float32)
    h = jnp.maximum(h + b1_ref[...], 0.0)
    xw = jnp.dot(h, wg_ref[...], preferred_element_type=jnp.float32)
    y_ref[...] = xw * _dis_from_views(d0_ref[...], d1_ref[...], RB)


def _ka(z_p, W1, b1r, Wg, deg0, deg1):
    return pl.pallas_call(
        _ka_body,
        grid=(GRID,),
        in_specs=[
            pl.BlockSpec((RB, D), lambda i: (i, 0)),
            pl.BlockSpec((D, D), lambda i: (0, 0)),
            pl.BlockSpec((1, D), lambda i: (0, 0)),
            pl.BlockSpec((D, D), lambda i: (0, 0)),
            pl.BlockSpec((RB // 8, D), lambda i: (i, 0)),
            pl.BlockSpec((RB // 8, D), lambda i: (i, 0)),
        ],
        out_specs=pl.BlockSpec((RB, D), lambda i: (i, 0)),
        out_shape=jax.ShapeDtypeStruct((NP, D), jnp.float32),
    )(z_p, W1, b1r, Wg, deg0, deg1)


@functools.partial(
    pl.kernel,
    out_type=jax.ShapeDtypeStruct((NP, D), jnp.float32),
    mesh=_mesh,
    compiler_params=_sc_params,
    scratch_types=[
        pltpu.VMEM((CW, CK), jnp.int32),
        pltpu.VMEM((CW, CK), jnp.int32),
        pltpu.VMEM((CK, HW), jnp.float32),
        pltpu.VMEM((CK, HW), jnp.float32),
        pltpu.VMEM((CK, HW), jnp.float32),
        pltpu.VMEM((CK, HW), jnp.float32),
        pltpu.VMEM((CK, HW), jnp.float32),
        pltpu.VMEM((CK, HW), jnp.float32),
        pltpu.VMEM((CK, HW), jnp.float32),
        pltpu.VMEM((CK, HW), jnp.float32),
        pltpu.VMEM_SHARED((NP, HW), jnp.float32),
        [pltpu.SemaphoreType.DMA] * 8,
        [pltpu.SemaphoreType.DMA] * 8,
    ],
)
def _seg_kernel(y2_hbm, s_hbm, d_hbm, zslab_hbm, p_hbm,
                sv, dv, rb0, rb1, rb2, rb3, rb4, rb5, rb6, rb7,
                acc, gs, sse):
    cid = lax.axis_index("c")
    sid = lax.axis_index("s")
    base = sid * RPT
    off = cid * HW

    pltpu.sync_copy(zslab_hbm, acc.at[pl.ds(base, RPT)])
    plsc.subcore_barrier()

    def _mkidx(r, carry):
        for c8 in range(CK // 16):
            sl = pl.ds(16 * c8, 16)
            sv[r, sl] = sv[r, sl] * 2 + cid
        return carry

    rbs = (rb0, rb1, rb2, rb3, rb4, rb5, rb6, rb7)
    NB = 8

    for half in range(2):
        wrow = sid * 2 + half
        pltpu.sync_copy(s_hbm.at[wrow], sv)
        pltpu.sync_copy(d_hbm.at[wrow], dv)
        lax.fori_loop(0, CW, _mkidx, 0)

        for b in range(NB // 2):
            pltpu.async_copy(y2_hbm.at[sv.at[b]], rbs[b], gs[b])
        for j in range(NB // 2):
            pltpu.make_async_copy(y2_hbm.at[sv.at[j]], rbs[j], gs[j]).wait()
            pltpu.async_copy(rbs[j], acc.at[dv.at[j]], sse[j], add=True)
            pltpu.async_copy(y2_hbm.at[sv.at[j + 4]], rbs[j + 4], gs[j + 4])

        def _body(t, carry):
            for b8 in range(NB):
                j = 4 + NB * t + b8
                bb = (4 + b8) % NB
                br = b8
                pltpu.make_async_copy(y2_hbm.at[sv.at[j]], rbs[bb], gs[bb]).wait()
                pltpu.async_copy(rbs[bb], acc.at[dv.at[j]], sse[bb], add=True)
                pltpu.make_async_copy(rbs[br], acc.at[dv.at[0]], sse[br]).wait()
                pltpu.async_copy(y2_hbm.at[sv.at[j + 4]], rbs[br], gs[br])
            return carry

        lax.fori_loop(0, (CW - 8) // NB, _body, 0)
        for j in (CW - 4, CW - 3, CW - 2, CW - 1):
            bb = j % NB
            pltpu.make_async_copy(y2_hbm.at[sv.at[j]], rbs[bb], gs[bb]).wait()
            pltpu.async_copy(rbs[bb], acc.at[dv.at[j]], sse[bb], add=True)
        for b in range(NB):
            pltpu.make_async_copy(rbs[b], acc.at[dv.at[0]], sse[b]).wait()

    plsc.subcore_barrier()
    pltpu.sync_copy(acc.at[pl.ds(base, RPT)],
                    p_hbm.at[pl.ds(base, RPT), pl.ds(off, HW)])


def _kc_body(p_ref, y_ref, d0_ref, d1_ref, bg_ref, w2_ref, b2_ref, o_ref):
    dis = _dis_from_views(d0_ref[...], d1_ref[...], RBO)
    h = jnp.maximum((p_ref[...] + y_ref[...]) * dis + bg_ref[...], 0.0)
    o = jnp.dot(h, w2_ref[...], preferred_element_type=jnp.float32)
    o = o + b2_ref[...]
    col = lax.broadcasted_iota(jnp.int32, (RBO, D), 1)
    o_ref[...] = jnp.where(col == 0, jax.nn.sigmoid(o), o)


def _kc(p, y, deg0, deg1, bgr, W2, b2r):
    return pl.pallas_call(
        _kc_body,
        grid=(GRID_O,),
        in_specs=[
            pl.BlockSpec((RBO, D), lambda i: (i, 0)),
            pl.BlockSpec((RBO, D), lambda i: (i, 0)),
            pl.BlockSpec((RBO // 8, D), lambda i: (i, 0)),
            pl.BlockSpec((RBO // 8, D), lambda i: (i, 0)),
            pl.BlockSpec((1, D), lambda i: (0, 0)),
            pl.BlockSpec((D, D), lambda i: (0, 0)),
            pl.BlockSpec((1, D), lambda i: (0, 0)),
        ],
        out_specs=pl.BlockSpec((RBO, D), lambda i: (i, 0)),
        out_shape=jax.ShapeDtypeStruct((N, D), jnp.float32),
    )(p, y, deg0, deg1, bgr, W2, b2r)


@jax.jit
def kernel(z, W1, b1, Wg, bg, W2, b2, edge_index):
    z_p = jnp.pad(z, ((0, NP - N), (0, 0)))
    b1r = b1.reshape(1, D)
    bgr = bg.reshape(1, D)
    b2r = b2.reshape(1, D)

    npad = EP - E
    pad_idx = (N + (jnp.arange(npad, dtype=jnp.int32) % (NP - N))).astype(jnp.int32)
    s_r = jnp.concatenate([edge_index[0], pad_idx]).reshape(NW, CW, CK)
    d_r = jnp.concatenate([edge_index[1], pad_idx]).reshape(NW, CW, CK)

    onehot = jnp.zeros((CK, 16), jnp.float32).at[:, 0].set(1.0)
    zeros16 = jnp.zeros((RPT, 16), jnp.float32)
    zslab = jnp.zeros((RPT, HW), jnp.float32)

    deg0, deg1 = _deg_kernel(d_r, onehot, zeros16)
    deg0v = deg0.reshape(NP // 8, D)
    deg1v = deg1.reshape(NP // 8, D)
    y = _ka(z_p, W1, b1r, Wg, deg0v, deg1v)
    y2 = y.reshape(2 * NP, HW)
    p = _seg_kernel(y2, s_r, d_r, zslab)
    return _kc(p, y, deg0v, deg1v, bgr, W2, b2r)
